# baseline (speedup 1.0000x reference)
import functools

import jax
import jax.numpy as jnp
from jax import lax
from jax.experimental import pallas as pl
from jax.experimental.pallas import tpu as pltpu
from jax.experimental.pallas import tpu_sc as plsc

_EPS = 1e-12
_CHUNK = 128
_L = 16


def _tree_add(vs):
    vs = list(vs)
    while len(vs) > 1:
        vs = [a + b for a, b in zip(vs[::2], vs[1::2])] + ([vs[-1]] if len(vs) % 2 else [])
    return vs[0]


def _hsum16(x):
    lane = lax.iota(jnp.int32, _L)
    dnums = lax.GatherDimensionNumbers(
        offset_dims=(), collapsed_slice_dims=(0,), start_index_map=(0,))
    for k in (1, 2, 4, 8):
        perm = lax.gather(x, (lane ^ k)[:, None], dnums, (1,),
                          mode=lax.GatherScatterMode.PROMISE_IN_BOUNDS)
        x = x + perm
    return x


def _rsqrt_vec(v):
    i = lax.bitcast_convert_type(v, jnp.int32)
    i = jnp.int32(0x5F3759DF) - lax.shift_right_logical(i, 1)
    y = lax.bitcast_convert_type(i, jnp.float32)
    for _ in range(3):
        y = y * (1.5 - 0.5 * v * y * y)
    return y


@functools.cache
def _make_fused(n_rows: int, emb: int, seq: int):
    info = plsc.get_sparse_core_info()
    nw = info.num_cores * info.num_subcores
    rows_per_w = n_rows // nw
    chunks = rows_per_w // _CHUNK
    pos_blocks = seq // rows_per_w
    nlane = emb // _L
    mesh = plsc.VectorSubcoreMesh(core_axis_name="c", subcore_axis_name="s")

    @functools.partial(
        pl.kernel,
        mesh=mesh,
        out_type=jax.ShapeDtypeStruct((n_rows, emb), jnp.float32),
        scratch_types=[
            pltpu.VMEM((chunks, _CHUNK), jnp.int32),
            pltpu.VMEM((rows_per_w, emb), jnp.float32),
            pltpu.VMEM((rows_per_w, emb), jnp.float32),
            pltpu.VMEM((1, emb), jnp.float32),
            pltpu.VMEM((1, emb), jnp.float32),
            pltpu.VMEM((1, emb), jnp.float32),
            pltpu.SemaphoreType.DMA,
            pltpu.SemaphoreType.DMA,
        ],
    )
    def fused(idx_hbm, table_hbm, pos_hbm, tok_hbm, g_hbm, b_hbm, out_hbm,
              idx_v, rows_v, bias_v, tok_v, g_v, b_v, gsem, osem):
        wid = lax.axis_index("s") * info.num_cores + lax.axis_index("c")
        base = wid * rows_per_w
        pltpu.sync_copy(idx_hbm.at[pl.ds(wid * chunks, chunks)], idx_v)
        gathers = [
            pltpu.async_copy(
                table_hbm.at[idx_v.at[j]],
                rows_v.at[pl.ds(j * _CHUNK, _CHUNK)],
                gsem,
            )
            for j in range(chunks)
        ]
        pos0 = (wid % pos_blocks) * rows_per_w
        pltpu.sync_copy(pos_hbm.at[pl.ds(pos0, rows_per_w)], bias_v)
        pltpu.sync_copy(tok_hbm.at[pl.ds(0, 1)], tok_v)
        pltpu.sync_copy(g_hbm, g_v)
        pltpu.sync_copy(b_hbm, b_v)
        tok = [tok_v[0, pl.ds(c * _L, _L)] for c in range(nlane)]
        gam = [g_v[0, pl.ds(c * _L, _L)] for c in range(nlane)]
        bet = [b_v[0, pl.ds(c * _L, _L)] for c in range(nlane)]
        inv_n = 1.0 / emb

        def row_body(r, carry):
            x = [
                rows_v[r, pl.ds(c * _L, _L)] + bias_v[r, pl.ds(c * _L, _L)] + tok[c]
                for c in range(nlane)
            ]
            mean = _hsum16(_tree_add(x)) * inv_n
            ex2 = _hsum16(_tree_add([v * v for v in x])) * inv_n
            inv = _rsqrt_vec(ex2 - mean * mean + _EPS)
            for c in range(nlane):
                rows_v[r, pl.ds(c * _L, _L)] = (x[c] - mean) * inv * gam[c] + bet[c]
            return carry

        outs = []
        for j in range(chunks):
            gathers[j].wait()
            lax.fori_loop(j * _CHUNK, (j + 1) * _CHUNK, row_body, 0)
            outs.append(
                pltpu.async_copy(
                    rows_v.at[pl.ds(j * _CHUNK, _CHUNK)],
                    out_hbm.at[pl.ds(base + j * _CHUNK, _CHUNK)],
                    osem,
                )
            )
        for cp in outs:
            cp.wait()

    return fused


def kernel(input_ids, word_embeddings, position_embeddings, token_type_embeddings, ln_gamma, ln_beta):
    batch, seq = input_ids.shape
    vocab, emb = word_embeddings.shape
    total = batch * seq
    ids = input_ids.astype(jnp.int32).reshape(total // _CHUNK, _CHUNK)
    out = _make_fused(total, emb, seq)(
        ids,
        word_embeddings,
        position_embeddings,
        token_type_embeddings,
        ln_gamma.reshape(1, emb),
        ln_beta.reshape(1, emb),
    )
    return out.reshape(batch, seq, emb)


# speedup vs baseline: 1.0680x; 1.0680x over previous
import functools

import jax
import jax.numpy as jnp
from jax import lax
from jax.experimental import pallas as pl
from jax.experimental.pallas import tpu as pltpu
from jax.experimental.pallas import tpu_sc as plsc

_EPS = 1e-12
_CHUNK = 128
_L = 16


def _tree_add(vs):
    vs = list(vs)
    while len(vs) > 1:
        vs = [a + b for a, b in zip(vs[::2], vs[1::2])] + ([vs[-1]] if len(vs) % 2 else [])
    return vs[0]


def _hsum16(x):
    lane = lax.iota(jnp.int32, _L)
    dnums = lax.GatherDimensionNumbers(
        offset_dims=(), collapsed_slice_dims=(0,), start_index_map=(0,))
    for k in (1, 2, 4, 8):
        perm = lax.gather(x, (lane ^ k)[:, None], dnums, (1,),
                          mode=lax.GatherScatterMode.PROMISE_IN_BOUNDS)
        x = x + perm
    return x


def _rsqrt_vec(v):
    i = lax.bitcast_convert_type(v, jnp.int32)
    i = jnp.int32(0x5F3759DF) - lax.shift_right_logical(i, 1)
    y = lax.bitcast_convert_type(i, jnp.float32)
    for _ in range(2):
        y = y * (1.5 - 0.5 * v * y * y)
    return y


@functools.cache
def _make_fused(n_rows: int, emb: int, seq: int):
    info = plsc.get_sparse_core_info()
    nw = info.num_cores * info.num_subcores
    rows_per_w = n_rows // nw
    chunks = rows_per_w // _CHUNK
    pos_blocks = seq // rows_per_w
    nlane = emb // _L
    mesh = plsc.VectorSubcoreMesh(core_axis_name="c", subcore_axis_name="s")

    @functools.partial(
        pl.kernel,
        mesh=mesh,
        out_type=jax.ShapeDtypeStruct((n_rows, emb), jnp.float32),
        scratch_types=[
            pltpu.VMEM((chunks, _CHUNK), jnp.int32),
            pltpu.VMEM((rows_per_w, emb), jnp.float32),
            pltpu.VMEM((rows_per_w, emb), jnp.float32),
            pltpu.VMEM((1, emb), jnp.float32),
            pltpu.VMEM((1, emb), jnp.float32),
            pltpu.VMEM((1, emb), jnp.float32),
            pltpu.SemaphoreType.DMA,
            pltpu.SemaphoreType.DMA,
        ],
    )
    def fused(idx_hbm, table_hbm, pos_hbm, tok_hbm, g_hbm, b_hbm, out_hbm,
              idx_v, rows_v, bias_v, tok_v, g_v, b_v, gsem, osem):
        wid = lax.axis_index("s") * info.num_cores + lax.axis_index("c")
        base = wid * rows_per_w
        pltpu.sync_copy(idx_hbm.at[pl.ds(wid * chunks, chunks)], idx_v)
        gathers = [
            pltpu.async_copy(
                table_hbm.at[idx_v.at[j]],
                rows_v.at[pl.ds(j * _CHUNK, _CHUNK)],
                gsem,
            )
            for j in range(chunks)
        ]
        pos0 = (wid % pos_blocks) * rows_per_w
        pltpu.sync_copy(pos_hbm.at[pl.ds(pos0, rows_per_w)], bias_v)
        pltpu.sync_copy(tok_hbm.at[pl.ds(0, 1)], tok_v)
        pltpu.sync_copy(g_hbm, g_v)
        pltpu.sync_copy(b_hbm, b_v)
        tok = [tok_v[0, pl.ds(c * _L, _L)] for c in range(nlane)]
        gam = [g_v[0, pl.ds(c * _L, _L)] for c in range(nlane)]
        bet = [b_v[0, pl.ds(c * _L, _L)] for c in range(nlane)]
        inv_n = 1.0 / emb

        def row_body(r):
            x = [
                rows_v[r, pl.ds(c * _L, _L)] + bias_v[r, pl.ds(c * _L, _L)] + tok[c]
                for c in range(nlane)
            ]
            mean = _hsum16(_tree_add(x)) * inv_n
            ex2 = _hsum16(_tree_add([v * v for v in x])) * inv_n
            inv = _rsqrt_vec(ex2 - mean * mean + _EPS)
            for c in range(nlane):
                rows_v[r, pl.ds(c * _L, _L)] = (x[c] - mean) * inv * gam[c] + bet[c]

        outs = []
        for j in range(chunks):
            gathers[j].wait()
            plsc.parallel_loop(j * _CHUNK, (j + 1) * _CHUNK, unroll=4)(row_body)
            outs.append(
                pltpu.async_copy(
                    rows_v.at[pl.ds(j * _CHUNK, _CHUNK)],
                    out_hbm.at[pl.ds(base + j * _CHUNK, _CHUNK)],
                    osem,
                )
            )
        for cp in outs:
            cp.wait()

    return fused


def kernel(input_ids, word_embeddings, position_embeddings, token_type_embeddings, ln_gamma, ln_beta):
    batch, seq = input_ids.shape
    vocab, emb = word_embeddings.shape
    total = batch * seq
    ids = input_ids.astype(jnp.int32).reshape(total // _CHUNK, _CHUNK)
    out = _make_fused(total, emb, seq)(
        ids,
        word_embeddings,
        position_embeddings,
        token_type_embeddings,
        ln_gamma.reshape(1, emb),
        ln_beta.reshape(1, emb),
    )
    return out.reshape(batch, seq, emb)
